# baseline (speedup 1.0000x reference)
import functools
import jax
import jax.numpy as jnp
from jax import lax
from jax.experimental import pallas as pl
from jax.experimental.pallas import tpu as pltpu
from jax.experimental.pallas import tpu_sc as plsc

E = 64
H = 1024
MOE_I = 512
SHARED_I = 4096
T = 64
SBLK = 512
NSH = SHARED_I // SBLK


def _dot_t(a, b):
    return jax.lax.dot_general(a, b, (((1,), (1,)), ((), ())),
                               preferred_element_type=jnp.float32)


def _router_body(x_ref, gw_ref, idx_ref):
    logits = _dot_t(x_ref[...], gw_ref[...])
    m = jnp.max(logits, axis=1, keepdims=True)
    eiota = jax.lax.broadcasted_iota(jnp.int32, (T, E), 1)
    cand = jnp.where(logits >= m, eiota, E)
    idx_ref[...] = jnp.min(cand, axis=1, keepdims=True)


def _shared_body(x_ref, sg_ref, su_ref, sd_ref, out_ref):
    j = pl.program_id(0)

    @pl.when(j == 0)
    def _():
        out_ref[...] = jnp.zeros_like(out_ref)

    x = x_ref[...]
    g = _dot_t(x, sg_ref[...])
    u = _dot_t(x, su_ref[...])
    act = jax.nn.silu(g) * u
    out_ref[...] += _dot_t(act, sd_ref[...])


def _moe_body(order_ref, n_ref, x_ref, top1_ref, shared_ref,
              wg_ref, wu_ref, wd_ref, out_ref):
    i = pl.program_id(0)

    @pl.when(i == 0)
    def _():
        out_ref[...] = shared_ref[...]

    @pl.when(i < n_ref[0])
    def _():
        e = order_ref[i]
        x = x_ref[...]
        g = _dot_t(x, wg_ref[0])
        u = _dot_t(x, wu_ref[0])
        act = jax.nn.silu(g) * u
        o = _dot_t(act, wd_ref[0])
        mask = (top1_ref[...] == e).astype(jnp.float32)
        out_ref[...] += o * mask


def _sc_compact(top1_flat):
    mesh = plsc.VectorSubcoreMesh(core_axis_name="c", subcore_axis_name="s")

    @functools.partial(
        pl.kernel, mesh=mesh,
        compiler_params=pltpu.CompilerParams(needs_layout_passes=False),
        out_type=[jax.ShapeDtypeStruct((E,), jnp.int32),
                  jax.ShapeDtypeStruct((16,), jnp.int32)],
        scratch_types=[pltpu.VMEM((T,), jnp.int32),
                       pltpu.VMEM((E,), jnp.int32),
                       pltpu.VMEM((E,), jnp.int32),
                       pltpu.VMEM((16,), jnp.int32)],
    )
    def body(top1_hbm, order_hbm, n_hbm, t_v, c_v, o_v, n_v):
        cidx = lax.axis_index("c")
        sidx = lax.axis_index("s")

        @pl.when((cidx == 0) & (sidx == 0))
        def _():
            pltpu.sync_copy(top1_hbm, t_v)
            zero16 = jnp.zeros((16,), jnp.int32)
            for k in range(E // 16):
                c_v[pl.ds(16 * k, 16)] = zero16
            ones16 = jnp.ones((16,), jnp.int32)
            for k in range(T // 16):
                tk = t_v[pl.ds(16 * k, 16)]
                plsc.store_scatter(c_v, [tk], ones16)
            carry = jnp.zeros((), jnp.int32)
            lastid = jnp.full((), -1, jnp.int32)
            for k in range(E // 16):
                ck = c_v[pl.ds(16 * k, 16)]
                ak = jnp.minimum(ck, 1)
                ps = plsc.cumsum(ak) + carry
                slotk = ps - 1
                eids = lax.broadcasted_iota(jnp.int32, (16,), 0) + 16 * k
                amask = ak > 0
                plsc.store_scatter(o_v, [slotk], eids, mask=amask)
                lastid = jnp.maximum(lastid, jnp.max(jnp.where(amask, eids, -1)))
                carry = carry + jnp.sum(ak)
            for k in range(E // 16):
                j16 = lax.broadcasted_iota(jnp.int32, (16,), 0) + 16 * k
                ov = o_v[pl.ds(16 * k, 16)]
                o_v[pl.ds(16 * k, 16)] = jnp.where(j16 < carry, ov, lastid)
            n_v[...] = jnp.zeros((16,), jnp.int32) + carry
            pltpu.sync_copy(o_v, order_hbm)
            pltpu.sync_copy(n_v, n_hbm)

    return body(top1_flat)


def kernel(hidden_states, gate_w, expert_gate_w, expert_up_w, expert_down_w,
           shared_gate_w, shared_up_w, shared_down_w):
    bsz, seq_len, hidden = hidden_states.shape
    x = hidden_states.reshape(T, H)

    top1 = pl.pallas_call(
        _router_body,
        out_shape=jax.ShapeDtypeStruct((T, 1), jnp.int32),
    )(x, gate_w)

    order, n8 = _sc_compact(top1.reshape(T))
    n = n8[:1]

    shared_out = pl.pallas_call(
        _shared_body,
        grid=(NSH,),
        in_specs=[
            pl.BlockSpec((T, H), lambda j: (0, 0)),
            pl.BlockSpec((SBLK, H), lambda j: (j, 0)),
            pl.BlockSpec((SBLK, H), lambda j: (j, 0)),
            pl.BlockSpec((H, SBLK), lambda j: (0, j)),
        ],
        out_specs=pl.BlockSpec((T, H), lambda j: (0, 0)),
        out_shape=jax.ShapeDtypeStruct((T, H), jnp.float32),
    )(x, shared_gate_w, shared_up_w, shared_down_w)

    out = pl.pallas_call(
        _moe_body,
        grid_spec=pltpu.PrefetchScalarGridSpec(
            num_scalar_prefetch=2,
            grid=(E,),
            in_specs=[
                pl.BlockSpec((T, H), lambda i, order, nn: (0, 0)),
                pl.BlockSpec((T, 1), lambda i, order, nn: (0, 0)),
                pl.BlockSpec((T, H), lambda i, order, nn: (0, 0)),
                pl.BlockSpec((1, MOE_I, H), lambda i, order, nn: (order[i], 0, 0)),
                pl.BlockSpec((1, MOE_I, H), lambda i, order, nn: (order[i], 0, 0)),
                pl.BlockSpec((1, H, MOE_I), lambda i, order, nn: (order[i], 0, 0)),
            ],
            out_specs=pl.BlockSpec((T, H), lambda i, order, nn: (0, 0)),
        ),
        out_shape=jax.ShapeDtypeStruct((T, H), jnp.float32),
    )(order, n, x, top1, shared_out,
      expert_gate_w, expert_up_w, expert_down_w)

    return out.reshape(bsz, seq_len, hidden)


# speedup vs baseline: 1.1367x; 1.1367x over previous
import jax
import jax.numpy as jnp
from jax.experimental import pallas as pl
from jax.experimental.pallas import tpu as pltpu

E = 64
H = 1024
MOE_I = 512
SHARED_I = 4096
T = 64
SBLK = 512
NSH = SHARED_I // SBLK


def _dot_t(a, b):
    return jax.lax.dot_general(a, b, (((1,), (1,)), ((), ())),
                               preferred_element_type=jnp.float32)


def _shared_body(x_ref, gw_ref, sg_ref, su_ref, sd_ref,
                 out_ref, top1_ref, order_ref, n_ref):
    j = pl.program_id(0)

    @pl.when(j == 0)
    def _():
        out_ref[...] = jnp.zeros_like(out_ref)
        x = x_ref[...]
        gw = gw_ref[...]
        lg = _dot_t(x, gw)
        m1 = jnp.max(lg, axis=1, keepdims=True)
        cand1 = jnp.where(lg >= m1, jax.lax.broadcasted_iota(jnp.int32, (T, E), 1), E)
        top1 = jnp.min(cand1, axis=1, keepdims=True)
        top1_ref[...] = top1
        oh = (top1 == jax.lax.broadcasted_iota(jnp.int32, (T, E), 1)
              ).astype(jnp.float32)
        ones_t = jnp.ones((T, 1), jnp.float32)
        counts = jax.lax.dot_general(oh, ones_t, (((0,), (0,)), ((), ())),
                                     preferred_element_type=jnp.float32)
        active = jnp.minimum(counts, 1.0)
        etri = (jax.lax.broadcasted_iota(jnp.int32, (E, E), 1)
                <= jax.lax.broadcasted_iota(jnp.int32, (E, E), 0)
                ).astype(jnp.float32)
        pos = jax.lax.dot_general(etri, active, (((1,), (0,)), ((), ())),
                                  preferred_element_type=jnp.float32)
        nact = jnp.sum(active, axis=0, keepdims=True)
        slot = pos - 1.0
        jio = jax.lax.broadcasted_iota(jnp.int32, (E, E), 1).astype(jnp.float32)
        order_oh = active * (slot == jio).astype(jnp.float32)
        evals = jax.lax.broadcasted_iota(jnp.int32, (E, 1), 0).astype(jnp.float32)
        order_row = jax.lax.dot_general(
            order_oh, evals, (((0,), (0,)), ((), ())),
            preferred_element_type=jnp.float32)
        lastid = jnp.max(evals * active, axis=0, keepdims=True)
        sio = jax.lax.broadcasted_iota(jnp.int32, (E, 1), 0).astype(jnp.float32)
        padded = jnp.where(sio < nact, order_row, lastid)
        order_ref[...] = padded.astype(jnp.int32)
        n_ref[...] = nact.astype(jnp.int32)

    x = x_ref[...]
    g = _dot_t(x, sg_ref[...])
    u = _dot_t(x, su_ref[...])
    act = jax.nn.silu(g) * u
    out_ref[...] += _dot_t(act, sd_ref[...])


def _moe_body(order_ref, n_ref, x_ref, top1_ref, shared_ref,
              wg_ref, wu_ref, wd_ref, out_ref):
    i = pl.program_id(0)

    @pl.when(i == 0)
    def _():
        out_ref[...] = shared_ref[...]

    @pl.when(i < n_ref[0])
    def _():
        e = order_ref[i]
        x = x_ref[...]
        g = _dot_t(x, wg_ref[0])
        u = _dot_t(x, wu_ref[0])
        act = jax.nn.silu(g) * u
        o = _dot_t(act, wd_ref[0])
        mask = (top1_ref[...] == e).astype(jnp.float32)
        out_ref[...] += o * mask


def kernel(hidden_states, gate_w, expert_gate_w, expert_up_w, expert_down_w,
           shared_gate_w, shared_up_w, shared_down_w):
    bsz, seq_len, hidden = hidden_states.shape
    x = hidden_states.reshape(T, H)

    shared_out, top1, order2d, n2d = pl.pallas_call(
        _shared_body,
        grid=(NSH,),
        in_specs=[
            pl.BlockSpec((T, H), lambda j: (0, 0)),
            pl.BlockSpec((E, H), lambda j: (0, 0)),
            pl.BlockSpec((SBLK, H), lambda j: (j, 0)),
            pl.BlockSpec((SBLK, H), lambda j: (j, 0)),
            pl.BlockSpec((H, SBLK), lambda j: (0, j)),
        ],
        out_specs=[
            pl.BlockSpec((T, H), lambda j: (0, 0)),
            pl.BlockSpec((T, 1), lambda j: (0, 0)),
            pl.BlockSpec((E, 1), lambda j: (0, 0)),
            pl.BlockSpec((1, 1), lambda j: (0, 0)),
        ],
        out_shape=[
            jax.ShapeDtypeStruct((T, H), jnp.float32),
            jax.ShapeDtypeStruct((T, 1), jnp.int32),
            jax.ShapeDtypeStruct((E, 1), jnp.int32),
            jax.ShapeDtypeStruct((1, 1), jnp.int32),
        ],
    )(x, gate_w, shared_gate_w, shared_up_w, shared_down_w)

    order = order2d.reshape(E)
    n = n2d.reshape(1)

    out = pl.pallas_call(
        _moe_body,
        grid_spec=pltpu.PrefetchScalarGridSpec(
            num_scalar_prefetch=2,
            grid=(E,),
            in_specs=[
                pl.BlockSpec((T, H), lambda i, order, nn: (0, 0)),
                pl.BlockSpec((T, 1), lambda i, order, nn: (0, 0)),
                pl.BlockSpec((T, H), lambda i, order, nn: (0, 0)),
                pl.BlockSpec((1, MOE_I, H), lambda i, order, nn: (order[i], 0, 0)),
                pl.BlockSpec((1, MOE_I, H), lambda i, order, nn: (order[i], 0, 0)),
                pl.BlockSpec((1, H, MOE_I), lambda i, order, nn: (order[i], 0, 0)),
            ],
            out_specs=pl.BlockSpec((T, H), lambda i, order, nn: (0, 0)),
        ),
        out_shape=jax.ShapeDtypeStruct((T, H), jnp.float32),
    )(order, n, x, top1, shared_out,
      expert_gate_w, expert_up_w, expert_down_w)

    return out.reshape(bsz, seq_len, hidden)
